# baseline (speedup 1.0000x reference)
import functools

import jax
import jax.numpy as jnp
from jax import lax
from jax.experimental import pallas as pl
from jax.experimental.pallas import tpu as pltpu
from jax.experimental.pallas import tpu_sc as plsc

HEIGHT, WIDTH, DIM, BATCH = 32, 32, 768, 64
LANES = 16
NC, NS = 2, 16
HALF = WIDTH // 2

_mesh = plsc.VectorSubcoreMesh(core_axis_name="c", subcore_axis_name="s")


@functools.partial(
    pl.kernel,
    mesh=_mesh,
    out_type=jax.ShapeDtypeStruct((BATCH, HEIGHT * WIDTH, DIM), jnp.float32),
    scratch_types=[
        pltpu.VMEM((WIDTH, DIM), jnp.float32),
        pltpu.VMEM((DIM,), jnp.float32),
        pltpu.SemaphoreType.DMA,
    ],
)
def _sc_pos_kernel(row_hbm, col_hbm, out_hbm, buf_v, row_v, sem):
    wid = lax.axis_index("s") * NC + lax.axis_index("c")

    with jax.named_scope("ph_input_dma"):
        ccol = pltpu.async_copy(col_hbm, buf_v, sem)
        crow = pltpu.async_copy(row_hbm.at[wid], row_v, sem)
        ccol.wait()
        crow.wait()

    def add_row(w, carry):
        for j in range(DIM // LANES):
            sl = pl.ds(j * LANES, LANES)
            buf_v[w, sl] = buf_v[w, sl] + row_v[sl]
        return carry

    base = wid * WIDTH
    group = 16
    pending = []

    def stream_half(lo):
        for g in range(BATCH // group):
            cur = [
                pltpu.async_copy(
                    buf_v.at[pl.ds(lo, HALF)],
                    out_hbm.at[b, pl.ds(base + lo, HALF)],
                    sem,
                )
                for b in range(g * group, (g + 1) * group)
            ]
            if pending:
                for c in pending.pop():
                    c.wait()
            pending.append(cur)

    with jax.named_scope("ph_compute1"):
        lax.fori_loop(0, HALF, add_row, 0)
    with jax.named_scope("ph_stream1"):
        stream_half(0)
    with jax.named_scope("ph_compute2"):
        lax.fori_loop(HALF, WIDTH, add_row, 0)
    with jax.named_scope("ph_stream2"):
        stream_half(HALF)
    with jax.named_scope("ph_drain"):
        while pending:
            for c in pending.pop():
                c.wait()


def kernel(batch_size, row_embed, col_embed):
    del batch_size
    return _sc_pos_kernel(row_embed, col_embed)


# speedup vs baseline: 1.0035x; 1.0035x over previous
import functools

import jax
import jax.numpy as jnp
from jax import lax
from jax.experimental import pallas as pl
from jax.experimental.pallas import tpu as pltpu
from jax.experimental.pallas import tpu_sc as plsc

HEIGHT, WIDTH, DIM, BATCH = 32, 32, 768, 64
LANES = 16
NC, NS = 2, 16
HALF = WIDTH // 2

_mesh = plsc.VectorSubcoreMesh(core_axis_name="c", subcore_axis_name="s")


@functools.partial(
    pl.kernel,
    mesh=_mesh,
    out_type=jax.ShapeDtypeStruct((BATCH, HEIGHT * WIDTH, DIM), jnp.float32),
    scratch_types=[
        pltpu.VMEM((WIDTH, DIM), jnp.float32),
        pltpu.VMEM((DIM,), jnp.float32),
        pltpu.SemaphoreType.DMA,
    ],
)
def _sc_pos_kernel(row_hbm, col_hbm, out_hbm, buf_v, row_v, sem):
    wid = lax.axis_index("s") * NC + lax.axis_index("c")
    c1 = pltpu.async_copy(col_hbm.at[pl.ds(0, HALF)], buf_v.at[pl.ds(0, HALF)], sem)
    c2 = pltpu.async_copy(col_hbm.at[pl.ds(HALF, HALF)], buf_v.at[pl.ds(HALF, HALF)], sem)
    crow = pltpu.async_copy(row_hbm.at[wid], row_v, sem)
    c1.wait()
    crow.wait()

    def add_row(w, carry):
        for j in range(DIM // LANES):
            sl = pl.ds(j * LANES, LANES)
            buf_v[w, sl] = buf_v[w, sl] + row_v[sl]
        return carry

    base = wid * WIDTH
    group = 16
    pending = []

    def stream_half(lo):
        for g in range(BATCH // group):
            cur = [
                pltpu.async_copy(
                    buf_v.at[pl.ds(lo, HALF)],
                    out_hbm.at[b, pl.ds(base + lo, HALF)],
                    sem,
                )
                for b in range(g * group, (g + 1) * group)
            ]
            if pending:
                for c in pending.pop():
                    c.wait()
            pending.append(cur)

    lax.fori_loop(0, HALF, add_row, 0)
    stream_half(0)
    c2.wait()
    lax.fori_loop(HALF, WIDTH, add_row, 0)
    stream_half(HALF)
    while pending:
        for c in pending.pop():
            c.wait()


def kernel(batch_size, row_embed, col_embed):
    del batch_size
    return _sc_pos_kernel(row_embed, col_embed)
